# baseline (speedup 1.0000x reference)
import functools

import jax
import jax.numpy as jnp
from jax import lax
from jax.experimental import pallas as pl
from jax.experimental.pallas import tpu as pltpu
from jax.experimental.pallas import tpu_sc as plsc

B = 4096
T = 200
DIM = 32
N = B * T
NC = 2
NS = 16
NW = NC * NS
PER_W = N // NW
G = 16
NG = PER_W // G
NBUF = 8

_mesh = plsc.VectorSubcoreMesh(core_axis_name="c", subcore_axis_name="s")


@functools.partial(
    pl.kernel,
    mesh=_mesh,
    out_type=jax.ShapeDtypeStruct((NW, NG, G, DIM), jnp.float32),
    compiler_params=pltpu.CompilerParams(use_tc_tiling_on_sc=False),
    scratch_types=[
        pltpu.VMEM((NG, G), jnp.int32),
        pltpu.VMEM((NBUF, G, DIM), jnp.float32),
    ]
    + [pltpu.SemaphoreType.DMA] * (2 * NBUF),
)
def _sc_gather(idx_hbm, table_hbm, out_hbm, idx_v, rows_v, *sems):
    gsem = sems[:NBUF]
    ssem = sems[NBUF:]
    wid = lax.axis_index("s") * NC + lax.axis_index("c")
    pltpu.sync_copy(idx_hbm.at[wid], idx_v)

    for b in range(NBUF):
        vec = idx_v[b, :]
        pltpu.async_copy(table_hbm.at[vec], rows_v.at[b], gsem[b])

    def body(j0, carry):
        for b in range(NBUF):
            j = j0 + b
            vec = idx_v[j, :]
            pltpu.make_async_copy(
                table_hbm.at[vec], rows_v.at[b], gsem[b]
            ).wait()
            pltpu.async_copy(rows_v.at[b], out_hbm.at[wid, j], ssem[b])

        for b in range(NBUF):
            j = j0 + b
            nxt = j + NBUF

            @pl.when(nxt < NG)
            def _():
                pltpu.make_async_copy(
                    rows_v.at[b], out_hbm.at[wid, j], ssem[b]
                ).wait()
                vec2 = idx_v[nxt, :]
                pltpu.async_copy(table_hbm.at[vec2], rows_v.at[b], gsem[b])

        return carry

    lax.fori_loop(0, NG // NBUF, lambda i, c: body(i * NBUF, c), 0)

    for b in range(NBUF):
        pltpu.make_async_copy(
            rows_v.at[b], out_hbm.at[wid, NG - NBUF + b], ssem[b]
        ).wait()


def kernel(x, table):
    idx = x.astype(jnp.int32).reshape(NW, NG, G)
    out = _sc_gather(idx, table)
    return out.reshape(B, T, DIM)


# speedup vs baseline: 1.1091x; 1.1091x over previous
import functools

import jax
import jax.numpy as jnp
from jax import lax
from jax.experimental import pallas as pl
from jax.experimental.pallas import tpu as pltpu
from jax.experimental.pallas import tpu_sc as plsc

B = 4096
T = 200
DIM = 32
N = B * T
NC = 2
NS = 16
NW = NC * NS
PER_W = N // NW
CH = 128
NCH = PER_W // CH
NBUF = 8

_mesh = plsc.VectorSubcoreMesh(core_axis_name="c", subcore_axis_name="s")


@functools.partial(
    pl.kernel,
    mesh=_mesh,
    out_type=jax.ShapeDtypeStruct((NW, NCH, CH, DIM), jnp.float32),
    compiler_params=pltpu.CompilerParams(use_tc_tiling_on_sc=False),
    scratch_types=[
        pltpu.VMEM((NCH, CH), jnp.int32),
        pltpu.VMEM((NBUF, CH, DIM), jnp.float32),
    ]
    + [pltpu.SemaphoreType.DMA] * (2 * NBUF),
)
def _sc_gather(idx_hbm, table_hbm, out_hbm, idx_v, rows_v, *sems):
    gsem = sems[:NBUF]
    ssem = sems[NBUF:]
    wid = lax.axis_index("s") * NC + lax.axis_index("c")
    pltpu.sync_copy(idx_hbm.at[wid], idx_v)

    for b in range(NBUF):
        pltpu.async_copy(table_hbm.at[idx_v.at[b]], rows_v.at[b], gsem[b])

    def body(j0, carry):
        for b in range(NBUF):
            j = j0 + b
            pltpu.make_async_copy(
                table_hbm.at[idx_v.at[j]], rows_v.at[b], gsem[b]
            ).wait()
            pltpu.async_copy(rows_v.at[b], out_hbm.at[wid, j], ssem[b])
            nxt = j + NBUF

            @pl.when(nxt < NCH)
            def _():
                pltpu.make_async_copy(
                    rows_v.at[b], out_hbm.at[wid, j], ssem[b]
                ).wait()
                pltpu.async_copy(table_hbm.at[idx_v.at[nxt]], rows_v.at[b], gsem[b])

        return carry

    lax.fori_loop(0, NCH // NBUF, lambda i, c: body(i * NBUF, c), 0)

    for b in range(NBUF):
        pltpu.make_async_copy(
            rows_v.at[b], out_hbm.at[wid, NCH - NBUF + b], ssem[b]
        ).wait()


def kernel(x, table):
    idx = x.astype(jnp.int32).reshape(NW, NCH, CH)
    out = _sc_gather(idx, table)
    return out.reshape(B, T, DIM)
